# baseline (speedup 1.0000x reference)
import jax
import jax.numpy as jnp
from jax.experimental import pallas as pl


def _add_kernel(tok_ref, pos_ref, out_ref):
    out_ref[...] = tok_ref[...] + pos_ref[...]


def kernel(x, token_embed, pos_table):
    B, L, D = token_embed.shape
    L_BLK = 512
    grid = (L // L_BLK,)
    return pl.pallas_call(
        _add_kernel,
        grid=grid,
        in_specs=[
            pl.BlockSpec((B, L_BLK, D), lambda i: (0, i, 0)),
            pl.BlockSpec((L_BLK, D), lambda i: (i, 0)),
        ],
        out_specs=pl.BlockSpec((B, L_BLK, D), lambda i: (0, i, 0)),
        out_shape=jax.ShapeDtypeStruct((B, L, D), token_embed.dtype),
    )(token_embed, pos_table)
